# Initial kernel scaffold; baseline (speedup 1.0000x reference)
import jax
import jax.numpy as jnp
from jax.experimental import pallas as pl


def kernel(x, pos, batch, edge_index, global_attr, params):
    raise NotImplementedError("write your pallas kernel here")


# speedup vs baseline: 2.0025x; 2.0025x over previous
import functools
import jax
import jax.numpy as jnp
from jax import lax
from jax.experimental import pallas as pl
from jax.experimental.pallas import tpu as pltpu
from jax.experimental.pallas import tpu_sc as plsc

NC = 2
NS = 16
NW = NC * NS
TW = 48
CH = 512
SUB = CH // 128


def _silu(v):
    return v * jax.nn.sigmoid(v)


def _make_gather(E, n_chunks):
    mesh = plsc.VectorSubcoreMesh(core_axis_name="c", subcore_axis_name="s")
    kmax = (n_chunks + NW - 1) // NW

    @functools.partial(
        pl.kernel,
        out_type=(jax.ShapeDtypeStruct((E, TW), jnp.float32),
                  jax.ShapeDtypeStruct((E, TW), jnp.float32)),
        mesh=mesh,
        compiler_params=pltpu.CompilerParams(use_tc_tiling_on_sc=False, needs_layout_passes=False),
        scratch_types=[
            pltpu.VMEM((CH,), jnp.int32),
            pltpu.VMEM((CH,), jnp.int32),
            pltpu.VMEM((CH, TW), jnp.float32),
            pltpu.VMEM((CH, TW), jnp.float32),
            pltpu.SemaphoreType.DMA,
        ],
    )
    def k(table, srcv, dstv, out_d, out_s, idx_s, idx_d, buf_s, buf_d, sem):
        cid = lax.axis_index("c")
        sid = lax.axis_index("s")
        wid = sid * NC + cid

        def body(kk, _):
            chunk = wid + kk * NW

            @pl.when(chunk < n_chunks)
            def _():
                base = chunk * CH
                pltpu.sync_copy(srcv.at[pl.ds(base, CH)], idx_s)
                pltpu.sync_copy(dstv.at[pl.ds(base, CH)], idx_d)
                descs = []
                for j in range(SUB):
                    descs.append(pltpu.async_copy(
                        table.at[idx_d.at[pl.ds(j * 128, 128)]],
                        buf_d.at[pl.ds(j * 128, 128)], sem))
                    descs.append(pltpu.async_copy(
                        table.at[idx_s.at[pl.ds(j * 128, 128)]],
                        buf_s.at[pl.ds(j * 128, 128)], sem))
                for d in descs:
                    d.wait()
                pltpu.sync_copy(buf_d, out_d.at[pl.ds(base, CH)])
                pltpu.sync_copy(buf_s, out_s.at[pl.ds(base, CH)])
            return ()

        lax.fori_loop(0, kmax, body, ())

    return k


def _make_scatter(E, N, n_chunks, mw):
    half = N // 2
    rows = ((half + NS) + NS - 1) // NS * NS
    rps = rows // NS
    mesh = plsc.VectorSubcoreMesh(core_axis_name="c", subcore_axis_name="s")
    kmax = (n_chunks + NS - 1) // NS

    @functools.partial(
        pl.kernel,
        out_type=jax.ShapeDtypeStruct((N, mw), jnp.float32),
        mesh=mesh,
        compiler_params=pltpu.CompilerParams(use_tc_tiling_on_sc=False, needs_layout_passes=False),
        scratch_types=[
            pltpu.VMEM((CH,), jnp.int32),
            pltpu.VMEM((SUB, 128), jnp.int32),
            pltpu.VMEM((CH, mw), jnp.float32),
            pltpu.VMEM_SHARED((rows, mw), jnp.float32),
        ],
    )
    def k(dstv, mv, zv, out_m, dbuf, idx2, mbuf, acc_m):
        cid = lax.axis_index("c")
        sid = lax.axis_index("s")
        nbase = cid * half

        pltpu.sync_copy(zv, acc_m.at[pl.ds(sid * rps, rps)])
        plsc.subcore_barrier()

        def body(kk, _):
            chunk = sid + kk * NS

            @pl.when(chunk < n_chunks)
            def _():
                base = chunk * CH
                pltpu.sync_copy(dstv.at[pl.ds(base, CH)], dbuf)
                pltpu.sync_copy(mv.at[pl.ds(base, CH)], mbuf)
                for g in range(CH // 16):
                    v = dbuf[pl.ds(g * 16, 16)] - nbase
                    ok = (v >= 0) & (v < half)
                    v = jnp.where(ok, v, half)
                    idx2[g // 8, pl.ds((g % 8) * 16, 16)] = v
                for j in range(SUB):
                    pltpu.sync_copy(mbuf.at[pl.ds(j * 128, 128)],
                                    acc_m.at[idx2.at[j]], add=True)
            return ()

        lax.fori_loop(0, kmax, body, ())
        plsc.subcore_barrier()

        wps = half // NS
        w0 = sid * wps
        pltpu.sync_copy(acc_m.at[pl.ds(w0, wps)],
                        out_m.at[pl.ds(nbase + w0, wps)])

    return k


def _make_segsum(NP, G):
    half = G // 2
    rows = half + 32
    zrows = rows // NS
    mesh = plsc.VectorSubcoreMesh(core_axis_name="c", subcore_axis_name="s")
    n_chunks = NP // CH
    kmax = (n_chunks + NS - 1) // NS
    rps = half // NS

    @functools.partial(
        pl.kernel,
        out_type=jax.ShapeDtypeStruct((G, TW), jnp.float32),
        mesh=mesh,
        compiler_params=pltpu.CompilerParams(use_tc_tiling_on_sc=False, needs_layout_passes=False),
        scratch_types=[
            pltpu.VMEM((CH,), jnp.int32),
            pltpu.VMEM((SUB, 128), jnp.int32),
            pltpu.VMEM((CH, TW), jnp.float32),
            pltpu.VMEM((zrows, TW), jnp.float32),
            pltpu.VMEM_SHARED((rows, TW), jnp.float32),
        ],
    )
    def k(bv, wv, out, bbuf, idx2, wbuf, zbuf, acc):
        cid = lax.axis_index("c")
        sid = lax.axis_index("s")
        gbase = cid * half

        zeros16 = jnp.zeros((16,), jnp.float32)
        for r in range(zrows):
            for c in range(TW // 16):
                zbuf[r, pl.ds(c * 16, 16)] = zeros16
        pltpu.sync_copy(zbuf, acc.at[pl.ds(sid * zrows, zrows)])
        plsc.subcore_barrier()

        def body(kk, _):
            chunk = sid + kk * NS

            @pl.when(chunk < n_chunks)
            def _():
                base = chunk * CH
                pltpu.sync_copy(bv.at[pl.ds(base, CH)], bbuf)
                pltpu.sync_copy(wv.at[pl.ds(base, CH)], wbuf)
                for g in range(CH // 16):
                    v = bbuf[pl.ds(g * 16, 16)] - gbase
                    ok = (v >= 0) & (v < half)
                    v = jnp.where(ok, v, half)
                    idx2[g // 8, pl.ds((g % 8) * 16, 16)] = v
                for j in range(SUB):
                    pltpu.sync_copy(wbuf.at[pl.ds(j * 128, 128)],
                                    acc.at[idx2.at[j]], add=True)
            return ()

        lax.fori_loop(0, kmax, body, ())
        plsc.subcore_barrier()
        pltpu.sync_copy(acc.at[pl.ds(sid * rps, rps)],
                        out.at[pl.ds(gbase + sid * rps, rps)])

    return k


def _emb_body(x_ref, pos_ref, w0, b0, w1, b1, t_ref):
    xb = x_ref[...]
    h = _silu(xb @ w0[...] + b0[...])
    h = h @ w1[...] + b1[...]
    phys = jnp.maximum(xb[:, 4:5], 1e-6)
    bn = xb.shape[0]
    t_ref[...] = jnp.concatenate(
        [h, pos_ref[...], phys, jnp.zeros((bn, TW - 36), jnp.float32)], axis=1)


def _edge_body(ad_ref, as_ref, w1, b1, w2, b2, wc1, bc1, wc2,
               m_ref, c_ref):
    ad = ad_ref[...]
    asr = as_ref[...]
    rel = ad[:, 32:35] - asr[:, 32:35]
    dist = jnp.sum(rel * rel, axis=1, keepdims=True)
    feats = jnp.concatenate([ad[:, 0:32], asr[:, 0:32], dist], axis=1)
    m1 = _silu(feats @ w1[...] + b1[...])
    m2 = _silu(m1 @ w2[...] + b2[...])
    cw = _silu(m2 @ wc1[...] + bc1[...])
    s = cw @ wc2[...]
    m_ref[...] = m2
    c_ref[...] = jnp.concatenate(
        [rel * s, jnp.zeros((ad.shape[0], 5), jnp.float32)], axis=1)


def _node_body(final, t_ref, am_ref, ac_ref, w1, b1, w2, b2, o_ref):
    t = t_ref[...]
    h = t[:, 0:32]
    phys = t[:, 35:36]
    u = _silu(jnp.concatenate([h, am_ref[...]], axis=1) @ w1[...] + b1[...])
    hn = h + (u @ w2[...] + b2[...])
    bn = t.shape[0]
    if final:
        o_ref[...] = jnp.concatenate(
            [hn * phys, phys, jnp.zeros((bn, TW - 33), jnp.float32)], axis=1)
    else:
        posn = t[:, 32:35] + ac_ref[...][:, 0:3]
        o_ref[...] = jnp.concatenate(
            [hn, posn, phys, jnp.zeros((bn, TW - 36), jnp.float32)], axis=1)


def _final_body(seg_ref, ga_ref, wg1, bg1, wg2, bg2, wh1, bh1, wh2, bh2,
                o_ref):
    seg = seg_ref[...]
    repr_ = seg[:, 0:32] / seg[:, 32:33]
    g = _silu(ga_ref[...] @ wg1[...] + bg1[...])
    g = g @ wg2[...] + bg2[...]
    comb = jnp.concatenate([repr_, g], axis=1)
    o = _silu(comb @ wh1[...] + bh1[...])
    o_ref[...] = o @ wh2[...] + bh2[...]


def _full(shape_nd):
    return pl.BlockSpec(shape_nd, lambda i: tuple(0 for _ in shape_nd))


def _rows(bs, w):
    return pl.BlockSpec((bs, w), lambda i: (i, 0))


def kernel(x, pos, batch, edge_index, global_attr, params):
    N, _ = x.shape
    E = edge_index.shape[1]
    G = global_attr.shape[0]
    src = edge_index[0].astype(jnp.int32)
    dst = edge_index[1].astype(jnp.int32)
    n_chunks = E // CH

    NP = ((N + CH - 1) // CH) * CH
    batch_pad = jnp.concatenate(
        [batch.astype(jnp.int32), jnp.full((NP - N,), G, jnp.int32)])

    p = params
    f32 = jnp.float32

    (we0, be0), (we1, be1) = p['emb'][0], p['emb'][1]
    BN = 1000
    T = pl.pallas_call(
        _emb_body,
        grid=(N // BN,),
        in_specs=[_rows(BN, x.shape[1]), _rows(BN, 3),
                  _full(we0.shape), _full((1, 32)),
                  _full(we1.shape), _full((1, 32))],
        out_specs=_rows(BN, TW),
        out_shape=jax.ShapeDtypeStruct((N, TW), f32),
    )(x, pos, we0, be0.reshape(1, -1), we1, be1.reshape(1, -1))

    gather = _make_gather(E, n_chunks)
    scatter_m = _make_scatter(E, N, n_chunks, 32)
    scatter_c = _make_scatter(E, N, n_chunks, 8)
    srows = (((N // 2 + NS) + NS - 1) // NS * NS) // NS
    z32 = jnp.zeros((srows, 32), f32)
    z8 = jnp.zeros((srows, 8), f32)
    BE = 512

    for l, cp in enumerate(p['convs']):
        final = (l == len(p['convs']) - 1)
        w_msg1, b_msg1 = cp['msg'][0]
        w2, b2 = cp['msg'][1]
        wc1, bc1 = cp['coord'][0]
        wc2 = cp['coord'][1][0]
        wn1, bn1 = cp['node'][0]
        wn2, bn2 = cp['node'][1]

        a_dst, a_src = gather(T, src, dst)

        m, coord = pl.pallas_call(
            _edge_body,
            grid=(E // BE,),
            in_specs=[_rows(BE, TW), _rows(BE, TW),
                      _full(w_msg1.shape),
                      _full((1, 32)), _full(w2.shape), _full((1, 32)),
                      _full(wc1.shape), _full((1, 32)), _full(wc2.shape)],
            out_specs=(_rows(BE, 32), _rows(BE, 8)),
            out_shape=(jax.ShapeDtypeStruct((E, 32), f32),
                       jax.ShapeDtypeStruct((E, 8), f32)),
        )(a_dst, a_src, w_msg1, b_msg1.reshape(1, -1), w2,
          b2.reshape(1, -1), wc1, bc1.reshape(1, -1), wc2)

        aggr_m = scatter_m(dst, m, z32)
        aggr_c = scatter_c(dst, coord, z8)

        out_rows = NP if final else N
        T = pl.pallas_call(
            functools.partial(_node_body, final),
            grid=(N // BN,),
            in_specs=[_rows(BN, TW), _rows(BN, 32), _rows(BN, 8),
                      _full(wn1.shape), _full((1, 32)),
                      _full(wn2.shape), _full((1, 32))],
            out_specs=_rows(BN, TW),
            out_shape=jax.ShapeDtypeStruct((out_rows, TW), f32),
        )(T, aggr_m, aggr_c, wn1, bn1.reshape(1, -1),
          wn2, bn2.reshape(1, -1))

    seg = _make_segsum(NP, G)(batch_pad, T)

    (wg1, bg1), (wg2, bg2) = p['glob'][0], p['glob'][1]
    (wh1, bh1), (wh2, bh2) = p['head'][0], p['head'][1]
    out = pl.pallas_call(
        _final_body,
        grid=(1,),
        in_specs=[_full((G, TW)), _full((G, 3)),
                  _full(wg1.shape), _full((1, 16)), _full(wg2.shape),
                  _full((1, 16)), _full(wh1.shape), _full((1, 32)),
                  _full(wh2.shape), _full((1, 1))],
        out_specs=_full((G, 1)),
        out_shape=jax.ShapeDtypeStruct((G, 1), f32),
    )(seg, global_attr, wg1, bg1.reshape(1, -1), wg2, bg2.reshape(1, -1),
      wh1, bh1.reshape(1, -1), wh2, bh2.reshape(1, -1))

    return out


# speedup vs baseline: 4.4490x; 2.2218x over previous
import functools
import jax
import jax.numpy as jnp
from jax import lax
from jax.experimental import pallas as pl
from jax.experimental.pallas import tpu as pltpu
from jax.experimental.pallas import tpu_sc as plsc

NC = 2
NS = 16
NW = NC * NS
TW = 48
CH = 512
SUB = CH // 128


def _silu(v):
    return v * jax.nn.sigmoid(v)


def _make_gather(E, n_chunks):
    mesh = plsc.VectorSubcoreMesh(core_axis_name="c", subcore_axis_name="s")
    kmax = (n_chunks + NW - 1) // NW

    @functools.partial(
        pl.kernel,
        out_type=jax.ShapeDtypeStruct((E, 128), jnp.float32),
        mesh=mesh,
        compiler_params=pltpu.CompilerParams(use_tc_tiling_on_sc=False, needs_layout_passes=False),
        scratch_types=[
            pltpu.VMEM((CH,), jnp.int32),
            pltpu.VMEM((CH,), jnp.int32),
            pltpu.VMEM((CH, TW), jnp.float32),
            pltpu.VMEM((CH, TW), jnp.float32),
            pltpu.SemaphoreType.DMA,
        ],
    )
    def k(table, srcv, dstv, out_a, idx_s, idx_d, buf_s, buf_d, sem):
        cid = lax.axis_index("c")
        sid = lax.axis_index("s")
        wid = sid * NC + cid

        def body(kk, _):
            chunk = wid + kk * NW

            @pl.when(chunk < n_chunks)
            def _():
                base = chunk * CH
                pltpu.sync_copy(srcv.at[pl.ds(base, CH)], idx_s)
                pltpu.sync_copy(dstv.at[pl.ds(base, CH)], idx_d)
                descs = []
                for j in range(SUB):
                    descs.append(pltpu.async_copy(
                        table.at[idx_d.at[pl.ds(j * 128, 128)]],
                        buf_d.at[pl.ds(j * 128, 128)], sem))
                    descs.append(pltpu.async_copy(
                        table.at[idx_s.at[pl.ds(j * 128, 128)]],
                        buf_s.at[pl.ds(j * 128, 128)], sem))
                for d in descs:
                    d.wait()
                pltpu.sync_copy(buf_d,
                                out_a.at[pl.ds(base, CH), pl.ds(0, TW)])
                pltpu.sync_copy(buf_s,
                                out_a.at[pl.ds(base, CH), pl.ds(TW, TW)])
            return ()

        lax.fori_loop(0, kmax, body, ())

    return k


def _make_scatter(E, N, n_chunks, mw, col0):
    half = N // 2
    rows = ((half + NS) + NS - 1) // NS * NS
    rps = rows // NS
    mesh = plsc.VectorSubcoreMesh(core_axis_name="c", subcore_axis_name="s")
    kmax = (n_chunks + NS - 1) // NS

    @functools.partial(
        pl.kernel,
        out_type=jax.ShapeDtypeStruct((N, mw), jnp.float32),
        mesh=mesh,
        compiler_params=pltpu.CompilerParams(use_tc_tiling_on_sc=False, needs_layout_passes=False),
        scratch_types=[
            pltpu.VMEM((CH,), jnp.int32),
            pltpu.VMEM((SUB, 128), jnp.int32),
            pltpu.VMEM((CH, mw), jnp.float32),
            pltpu.VMEM_SHARED((rows, mw), jnp.float32),
        ],
    )
    def k(dstv, mv, zv, out_m, dbuf, idx2, mbuf, acc_m):
        cid = lax.axis_index("c")
        sid = lax.axis_index("s")
        nbase = cid * half

        pltpu.sync_copy(zv, acc_m.at[pl.ds(sid * rps, rps)])
        plsc.subcore_barrier()

        def body(kk, _):
            chunk = sid + kk * NS

            @pl.when(chunk < n_chunks)
            def _():
                base = chunk * CH
                pltpu.sync_copy(dstv.at[pl.ds(base, CH)], dbuf)
                pltpu.sync_copy(mv.at[pl.ds(base, CH), pl.ds(col0, mw)],
                                mbuf)
                for g in range(CH // 16):
                    v = dbuf[pl.ds(g * 16, 16)] - nbase
                    ok = (v >= 0) & (v < half)
                    v = jnp.where(ok, v, half)
                    idx2[g // 8, pl.ds((g % 8) * 16, 16)] = v
                for j in range(SUB):
                    pltpu.sync_copy(mbuf.at[pl.ds(j * 128, 128)],
                                    acc_m.at[idx2.at[j]], add=True)
            return ()

        lax.fori_loop(0, kmax, body, ())
        plsc.subcore_barrier()

        wps = half // NS
        w0 = sid * wps
        pltpu.sync_copy(acc_m.at[pl.ds(w0, wps)],
                        out_m.at[pl.ds(nbase + w0, wps)])

    return k


def _make_segsum(NP, G):
    half = G // 2
    rows = half + 32
    zrows = rows // NS
    mesh = plsc.VectorSubcoreMesh(core_axis_name="c", subcore_axis_name="s")
    n_chunks = NP // CH
    kmax = (n_chunks + NS - 1) // NS
    rps = half // NS

    @functools.partial(
        pl.kernel,
        out_type=jax.ShapeDtypeStruct((G, TW), jnp.float32),
        mesh=mesh,
        compiler_params=pltpu.CompilerParams(use_tc_tiling_on_sc=False, needs_layout_passes=False),
        scratch_types=[
            pltpu.VMEM((CH,), jnp.int32),
            pltpu.VMEM((SUB, 128), jnp.int32),
            pltpu.VMEM((CH, TW), jnp.float32),
            pltpu.VMEM((zrows, TW), jnp.float32),
            pltpu.VMEM_SHARED((rows, TW), jnp.float32),
        ],
    )
    def k(bv, wv, out, bbuf, idx2, wbuf, zbuf, acc):
        cid = lax.axis_index("c")
        sid = lax.axis_index("s")
        gbase = cid * half

        zeros16 = jnp.zeros((16,), jnp.float32)
        for r in range(zrows):
            for c in range(TW // 16):
                zbuf[r, pl.ds(c * 16, 16)] = zeros16
        pltpu.sync_copy(zbuf, acc.at[pl.ds(sid * zrows, zrows)])
        plsc.subcore_barrier()

        def body(kk, _):
            chunk = sid + kk * NS

            @pl.when(chunk < n_chunks)
            def _():
                base = chunk * CH
                pltpu.sync_copy(bv.at[pl.ds(base, CH)], bbuf)
                pltpu.sync_copy(wv.at[pl.ds(base, CH)], wbuf)
                for g in range(CH // 16):
                    v = bbuf[pl.ds(g * 16, 16)] - gbase
                    ok = (v >= 0) & (v < half)
                    v = jnp.where(ok, v, half)
                    idx2[g // 8, pl.ds((g % 8) * 16, 16)] = v
                for j in range(SUB):
                    pltpu.sync_copy(wbuf.at[pl.ds(j * 128, 128)],
                                    acc.at[idx2.at[j]], add=True)
            return ()

        lax.fori_loop(0, kmax, body, ())
        plsc.subcore_barrier()
        pltpu.sync_copy(acc.at[pl.ds(sid * rps, rps)],
                        out.at[pl.ds(gbase + sid * rps, rps)])

    return k


def _emb_body(x_ref, pos_ref, w0, b0, w1, b1, t_ref):
    xb = x_ref[...]
    h = _silu(xb @ w0[...] + b0[...])
    h = h @ w1[...] + b1[...]
    phys = jnp.maximum(xb[:, 4:5], 1e-6)
    bn = xb.shape[0]
    t_ref[...] = jnp.concatenate(
        [h, pos_ref[...], phys, jnp.zeros((bn, TW - 36), jnp.float32)], axis=1)


def _edge_body(a_ref, w1, b1, w2, b2, wc1, bc1, wc2, mc_ref):
    a = a_ref[...]
    rel = a[:, 32:35] - a[:, 80:83]
    dist = jnp.sum(rel * rel, axis=1, keepdims=True)
    feats = jnp.concatenate([a[:, 0:32], a[:, 48:80], dist], axis=1)
    m1 = _silu(feats @ w1[...] + b1[...])
    m2 = _silu(m1 @ w2[...] + b2[...])
    cw = _silu(m2 @ wc1[...] + bc1[...])
    s = cw @ wc2[...]
    mc_ref[...] = jnp.concatenate(
        [m2, rel * s, jnp.zeros((a.shape[0], 93), jnp.float32)], axis=1)


def _node_body(final, t_ref, am_ref, ac_ref, w1, b1, w2, b2, o_ref):
    t = t_ref[...]
    h = t[:, 0:32]
    phys = t[:, 35:36]
    u = _silu(jnp.concatenate([h, am_ref[...]], axis=1) @ w1[...] + b1[...])
    hn = h + (u @ w2[...] + b2[...])
    bn = t.shape[0]
    if final:
        o_ref[...] = jnp.concatenate(
            [hn * phys, phys, jnp.zeros((bn, TW - 33), jnp.float32)], axis=1)
    else:
        posn = t[:, 32:35] + ac_ref[...][:, 0:3]
        o_ref[...] = jnp.concatenate(
            [hn, posn, phys, jnp.zeros((bn, TW - 36), jnp.float32)], axis=1)


def _final_body(seg_ref, ga_ref, wg1, bg1, wg2, bg2, wh1, bh1, wh2, bh2,
                o_ref):
    seg = seg_ref[...]
    repr_ = seg[:, 0:32] / seg[:, 32:33]
    g = _silu(ga_ref[...] @ wg1[...] + bg1[...])
    g = g @ wg2[...] + bg2[...]
    comb = jnp.concatenate([repr_, g], axis=1)
    o = _silu(comb @ wh1[...] + bh1[...])
    o_ref[...] = o @ wh2[...] + bh2[...]


def _full(shape_nd):
    return pl.BlockSpec(shape_nd, lambda i: tuple(0 for _ in shape_nd))


def _rows(bs, w):
    return pl.BlockSpec((bs, w), lambda i: (i, 0))


def kernel(x, pos, batch, edge_index, global_attr, params):
    N, _ = x.shape
    E = edge_index.shape[1]
    G = global_attr.shape[0]
    src = edge_index[0].astype(jnp.int32)
    dst = edge_index[1].astype(jnp.int32)
    n_chunks = E // CH

    NP = ((N + CH - 1) // CH) * CH
    batch_pad = jnp.concatenate(
        [batch.astype(jnp.int32), jnp.full((NP - N,), G, jnp.int32)])

    p = params
    f32 = jnp.float32

    (we0, be0), (we1, be1) = p['emb'][0], p['emb'][1]
    BN = 1000
    T = pl.pallas_call(
        _emb_body,
        grid=(N // BN,),
        in_specs=[_rows(BN, x.shape[1]), _rows(BN, 3),
                  _full(we0.shape), _full((1, 32)),
                  _full(we1.shape), _full((1, 32))],
        out_specs=_rows(BN, TW),
        out_shape=jax.ShapeDtypeStruct((N, TW), f32),
    )(x, pos, we0, be0.reshape(1, -1), we1, be1.reshape(1, -1))

    gather = _make_gather(E, n_chunks)
    scatter_m = _make_scatter(E, N, n_chunks, 32, 0)
    scatter_c = _make_scatter(E, N, n_chunks, 8, 32)
    srows = (((N // 2 + NS) + NS - 1) // NS * NS) // NS
    z32 = jnp.zeros((srows, 32), f32)
    z8 = jnp.zeros((srows, 8), f32)
    BE = 3200

    for l, cp in enumerate(p['convs']):
        final = (l == len(p['convs']) - 1)
        w_msg1, b_msg1 = cp['msg'][0]
        w2, b2 = cp['msg'][1]
        wc1, bc1 = cp['coord'][0]
        wc2 = cp['coord'][1][0]
        wn1, bn1 = cp['node'][0]
        wn2, bn2 = cp['node'][1]

        a_pack = gather(T, src, dst)

        mc = pl.pallas_call(
            _edge_body,
            grid=(E // BE,),
            in_specs=[_rows(BE, 128),
                      _full(w_msg1.shape),
                      _full((1, 32)), _full(w2.shape), _full((1, 32)),
                      _full(wc1.shape), _full((1, 32)), _full(wc2.shape)],
            out_specs=_rows(BE, 128),
            out_shape=jax.ShapeDtypeStruct((E, 128), f32),
        )(a_pack, w_msg1, b_msg1.reshape(1, -1), w2,
          b2.reshape(1, -1), wc1, bc1.reshape(1, -1), wc2)

        aggr_m = scatter_m(dst, mc, z32)
        aggr_c = scatter_c(dst, mc, z8)

        out_rows = NP if final else N
        T = pl.pallas_call(
            functools.partial(_node_body, final),
            grid=(N // BN,),
            in_specs=[_rows(BN, TW), _rows(BN, 32), _rows(BN, 8),
                      _full(wn1.shape), _full((1, 32)),
                      _full(wn2.shape), _full((1, 32))],
            out_specs=_rows(BN, TW),
            out_shape=jax.ShapeDtypeStruct((out_rows, TW), f32),
        )(T, aggr_m, aggr_c, wn1, bn1.reshape(1, -1),
          wn2, bn2.reshape(1, -1))

    seg = _make_segsum(NP, G)(batch_pad, T)

    (wg1, bg1), (wg2, bg2) = p['glob'][0], p['glob'][1]
    (wh1, bh1), (wh2, bh2) = p['head'][0], p['head'][1]
    out = pl.pallas_call(
        _final_body,
        grid=(1,),
        in_specs=[_full((G, TW)), _full((G, 3)),
                  _full(wg1.shape), _full((1, 16)), _full(wg2.shape),
                  _full((1, 16)), _full(wh1.shape), _full((1, 32)),
                  _full(wh2.shape), _full((1, 1))],
        out_specs=_full((G, 1)),
        out_shape=jax.ShapeDtypeStruct((G, 1), f32),
    )(seg, global_attr, wg1, bg1.reshape(1, -1), wg2, bg2.reshape(1, -1),
      wh1, bh1.reshape(1, -1), wh2, bh2.reshape(1, -1))

    return out
